# baseline (speedup 1.0000x reference)
import functools
import math

import jax
import jax.numpy as jnp
from jax import lax
from jax.experimental import pallas as pl
from jax.experimental.pallas import tpu as pltpu
from jax.experimental.pallas import tpu_sc as plsc

EMB = 64
LANES = 16
IDXW = 128
K = 4
CHUNK = IDXW * K


@functools.lru_cache(maxsize=None)
def _build(n_rows: int, n_words: int):
    info = plsc.get_sparse_core_info()
    nc, ns = info.num_cores, info.num_subcores
    nw = nc * ns
    rpw = n_rows // nw
    chunks = rpw // CHUNK
    assert rpw % CHUNK == 0

    mesh = plsc.VectorSubcoreMesh(core_axis_name="c", subcore_axis_name="s")

    @functools.partial(
        pl.kernel,
        out_type=jax.ShapeDtypeStruct((n_rows, EMB), jnp.float32),
        mesh=mesh,
        scratch_types=[
            pltpu.VMEM((K, IDXW), jnp.int32),
            pltpu.VMEM((CHUNK, EMB), jnp.float32),
            pltpu.VMEM((EMB,), jnp.float32),
            pltpu.SemaphoreType.DMA,
        ],
        compiler_params=pltpu.CompilerParams(use_tc_tiling_on_sc=False),
    )
    def emb_kernel(tok_hbm, table_hbm, pos_hbm, out_hbm, idx_v, rows_v, pos_v, gsem):
        wid = lax.axis_index("s") * nc + lax.axis_index("c")
        base_row = wid * rpw
        base128 = wid * (rpw // IDXW)

        pltpu.sync_copy(pos_hbm, pos_v)
        p = [pos_v[pl.ds(q * LANES, LANES)] for q in range(EMB // LANES)]

        def chunk_body(g, carry):
            row0 = base_row + g * CHUNK
            pltpu.sync_copy(tok_hbm.at[pl.ds(base128 + g * K, K)], idx_v)
            cps = [
                pltpu.async_copy(
                    table_hbm.at[idx_v.at[j]],
                    rows_v.at[pl.ds(j * IDXW, IDXW)],
                    gsem,
                )
                for j in range(K)
            ]
            for cp in cps:
                cp.wait()

            def row_body(i, c):
                for q in range(EMB // LANES):
                    sl = (i, pl.ds(q * LANES, LANES))
                    rows_v[sl] = rows_v[sl] + p[q]
                return c

            lax.fori_loop(0, CHUNK, row_body, 0)
            pltpu.sync_copy(rows_v, out_hbm.at[pl.ds(row0, CHUNK)])
            return carry

        lax.fori_loop(0, chunks, chunk_body, 0)

    return emb_kernel


def kernel(tokens, table):
    t_dim, b_dim = tokens.shape
    n_rows = t_dim * b_dim
    n_words, emb = table.shape

    den = jnp.exp(-jnp.arange(0, emb, 2, dtype=jnp.float32) * math.log(10000.0) / emb)
    pos = jnp.zeros((emb,), dtype=jnp.float32)
    pos = pos.at[0::2].set(jnp.sin(t_dim * den))
    pos = pos.at[1::2].set(jnp.cos(t_dim * den))

    tok = tokens.reshape(n_rows // IDXW, IDXW).astype(jnp.int32)
    out = _build(n_rows, n_words)(tok, table, pos)
    return out.reshape(t_dim, b_dim, emb)


# speedup vs baseline: 1.1366x; 1.1366x over previous
import functools
import math

import jax
import jax.numpy as jnp
from jax import lax
from jax.experimental import pallas as pl
from jax.experimental.pallas import tpu as pltpu
from jax.experimental.pallas import tpu_sc as plsc

EMB = 64
LANES = 16
CHUNK = 128
NBUF = 8
AHEAD = NBUF - 2


@functools.lru_cache(maxsize=None)
def _build(n_rows: int, n_words: int):
    info = plsc.get_sparse_core_info()
    nc, ns = info.num_cores, info.num_subcores
    nw = nc * ns
    rpw = n_rows // nw
    chunks = rpw // CHUNK
    assert rpw % CHUNK == 0 and chunks % NBUF == 0
    outer = chunks // NBUF

    mesh = plsc.VectorSubcoreMesh(core_axis_name="c", subcore_axis_name="s")

    @functools.partial(
        pl.kernel,
        out_type=jax.ShapeDtypeStruct((n_rows, EMB), jnp.float32),
        mesh=mesh,
        scratch_types=[
            pltpu.VMEM((chunks, CHUNK), jnp.int32),
            pltpu.VMEM((NBUF, CHUNK, EMB), jnp.float32),
            pltpu.VMEM((EMB,), jnp.float32),
            pltpu.SemaphoreType.DMA((NBUF,)),
            pltpu.SemaphoreType.DMA((NBUF,)),
        ],
        compiler_params=pltpu.CompilerParams(use_tc_tiling_on_sc=False),
    )
    def emb_kernel(tok_hbm, table_hbm, pos_hbm, out_hbm, idx_all, bufs, pos_v,
                   gsem, ssem):
        wid = lax.axis_index("s") * nc + lax.axis_index("c")
        base_row = wid * rpw
        base128 = wid * chunks

        pltpu.sync_copy(pos_hbm, pos_v)
        p = [pos_v[pl.ds(q * LANES, LANES)] for q in range(EMB // LANES)]
        pltpu.sync_copy(tok_hbm.at[pl.ds(base128, chunks)], idx_all)

        def gather(c, b):
            pltpu.async_copy(table_hbm.at[idx_all.at[c]], bufs.at[b], gsem.at[b])

        def gather_wait(c, b):
            pltpu.make_async_copy(
                table_hbm.at[idx_all.at[c]], bufs.at[b], gsem.at[b]).wait()

        def store(c, b):
            pltpu.async_copy(
                bufs.at[b], out_hbm.at[pl.ds(base_row + c * CHUNK, CHUNK)],
                ssem.at[b])

        def store_wait(b):
            pltpu.make_async_copy(
                bufs.at[b], out_hbm.at[pl.ds(base_row, CHUNK)],
                ssem.at[b]).wait()

        for f in range(AHEAD):
            gather(f, f)

        def outer_body(o, carry):
            for b in range(NBUF):
                t = o * NBUF + b
                f = t + AHEAD
                fb = (b + AHEAD) % NBUF

                @pl.when(t > 1)
                def _wait_store():
                    store_wait(fb)

                @pl.when(f < chunks)
                def _issue_gather():
                    gather(f, fb)

                gather_wait(t, b)

                def row_body(i, c):
                    for q in range(EMB // LANES):
                        sl = (b, i, pl.ds(q * LANES, LANES))
                        bufs[sl] = bufs[sl] + p[q]
                    return c

                lax.fori_loop(0, CHUNK, row_body, 0)
                store(t, b)
            return carry

        lax.fori_loop(0, outer, outer_body, 0)

        for k in range(chunks - (NBUF - AHEAD), chunks):
            store_wait(k % NBUF)

    return emb_kernel


def kernel(tokens, table):
    t_dim, b_dim = tokens.shape
    n_rows = t_dim * b_dim
    n_words, emb = table.shape

    den = jnp.exp(-jnp.arange(0, emb, 2, dtype=jnp.float32) * math.log(10000.0) / emb)
    pos = jnp.zeros((emb,), dtype=jnp.float32)
    pos = pos.at[0::2].set(jnp.sin(t_dim * den))
    pos = pos.at[1::2].set(jnp.cos(t_dim * den))

    tok = tokens.reshape(n_rows // CHUNK, CHUNK).astype(jnp.int32)
    out = _build(n_rows, n_words)(tok, table, pos)
    return out.reshape(t_dim, b_dim, emb)


# speedup vs baseline: 1.3970x; 1.2291x over previous
import functools
import math

import jax
import jax.numpy as jnp
from jax import lax
from jax.experimental import pallas as pl
from jax.experimental.pallas import tpu as pltpu
from jax.experimental.pallas import tpu_sc as plsc

EMB = 64
ROW = 128
LANES = 16
CHUNK = 64
NBUF = 8
AHEAD = NBUF - 2


@functools.lru_cache(maxsize=None)
def _build(n_rows: int, n_words: int):
    info = plsc.get_sparse_core_info()
    nc, ns = info.num_cores, info.num_subcores
    nw = nc * ns
    rpw = n_rows // nw
    chunks = rpw // CHUNK
    assert rpw % CHUNK == 0 and chunks % NBUF == 0
    outer = chunks // NBUF
    idx_rows = rpw // 128

    mesh = plsc.VectorSubcoreMesh(core_axis_name="c", subcore_axis_name="s")

    @functools.partial(
        pl.kernel,
        out_type=jax.ShapeDtypeStruct((n_rows, ROW), jnp.float32),
        mesh=mesh,
        scratch_types=[
            pltpu.VMEM((idx_rows, 128), jnp.int32),
            pltpu.VMEM((NBUF, CHUNK, ROW), jnp.float32),
            pltpu.VMEM((EMB,), jnp.float32),
            pltpu.SemaphoreType.DMA((NBUF,)),
            pltpu.SemaphoreType.DMA((NBUF,)),
        ],
    )
    def emb_kernel(tok_hbm, table_hbm, pos_hbm, out_hbm, idx_all, bufs, pos_v,
                   gsem, ssem):
        wid = lax.axis_index("s") * nc + lax.axis_index("c")
        base_row = wid * rpw
        base128 = wid * idx_rows

        pltpu.sync_copy(pos_hbm, pos_v)
        p = [pos_v[pl.ds(q * LANES, LANES)] for q in range(EMB // LANES)]
        pltpu.sync_copy(tok_hbm.at[pl.ds(base128, idx_rows)], idx_all)

        def idx_slice(c):
            return idx_all.at[c // 2, pl.ds((c % 2) * CHUNK, CHUNK)]

        def gather(c, b):
            pltpu.async_copy(table_hbm.at[idx_slice(c)], bufs.at[b], gsem.at[b])

        def gather_wait(c, b):
            pltpu.make_async_copy(
                table_hbm.at[idx_slice(c)], bufs.at[b], gsem.at[b]).wait()

        def store(c, b):
            pltpu.async_copy(
                bufs.at[b], out_hbm.at[pl.ds(base_row + c * CHUNK, CHUNK)],
                ssem.at[b])

        def store_wait(b):
            pltpu.make_async_copy(
                bufs.at[b], out_hbm.at[pl.ds(base_row, CHUNK)],
                ssem.at[b]).wait()

        for f in range(AHEAD):
            gather(f, f)

        def outer_body(o, carry):
            for b in range(NBUF):
                t = o * NBUF + b
                f = t + AHEAD
                fb = (b + AHEAD) % NBUF

                @pl.when(t > 1)
                def _wait_store():
                    store_wait(fb)

                @pl.when(f < chunks)
                def _issue_gather():
                    gather(f, fb)

                gather_wait(t, b)

                def row_body(i, c):
                    for q in range(EMB // LANES):
                        sl = (b, i, pl.ds(q * LANES, LANES))
                        bufs[sl] = bufs[sl] + p[q]
                    return c

                lax.fori_loop(0, CHUNK, row_body, 0)
                store(t, b)
            return carry

        lax.fori_loop(0, outer, outer_body, 0)

        for k in range(chunks - (NBUF - AHEAD), chunks):
            store_wait(k % NBUF)

    return emb_kernel


def kernel(tokens, table):
    t_dim, b_dim = tokens.shape
    n_rows = t_dim * b_dim
    n_words, emb = table.shape

    den = jnp.exp(-jnp.arange(0, emb, 2, dtype=jnp.float32) * math.log(10000.0) / emb)
    pos = jnp.zeros((emb,), dtype=jnp.float32)
    pos = pos.at[0::2].set(jnp.sin(t_dim * den))
    pos = pos.at[1::2].set(jnp.cos(t_dim * den))

    tok = tokens.reshape(n_rows // 128, 128).astype(jnp.int32)
    table_p = jnp.pad(table, ((0, 0), (0, ROW - emb)))
    out = _build(n_rows, n_words)(tok, table_p, pos)
    return out[:, :emb].reshape(t_dim, b_dim, emb)


# speedup vs baseline: 1.4021x; 1.0036x over previous
import functools
import math

import jax
import jax.numpy as jnp
from jax import lax
from jax.experimental import pallas as pl
from jax.experimental.pallas import tpu as pltpu
from jax.experimental.pallas import tpu_sc as plsc

EMB = 64
ROW = 128
LANES = 16
CHUNK = 64
NBUF = 10
AHEAD = NBUF - 2


@functools.lru_cache(maxsize=None)
def _build(n_rows: int, n_words: int):
    info = plsc.get_sparse_core_info()
    nc, ns = info.num_cores, info.num_subcores
    nw = nc * ns
    rpw = n_rows // nw
    chunks = rpw // CHUNK
    assert rpw % CHUNK == 0 and chunks % NBUF == 0
    outer = chunks // NBUF
    idx_rows = rpw // 128

    mesh = plsc.VectorSubcoreMesh(core_axis_name="c", subcore_axis_name="s")

    @functools.partial(
        pl.kernel,
        out_type=jax.ShapeDtypeStruct((n_rows, ROW), jnp.float32),
        mesh=mesh,
        scratch_types=[
            pltpu.VMEM((idx_rows, 128), jnp.int32),
            pltpu.VMEM((NBUF, CHUNK, ROW), jnp.float32),
            pltpu.VMEM((EMB,), jnp.float32),
            pltpu.SemaphoreType.DMA((NBUF,)),
            pltpu.SemaphoreType.DMA((NBUF,)),
        ],
    )
    def emb_kernel(tok_hbm, table_hbm, pos_hbm, out_hbm, idx_all, bufs, pos_v,
                   gsem, ssem):
        wid = lax.axis_index("s") * nc + lax.axis_index("c")
        base_row = wid * rpw
        base128 = wid * idx_rows

        pltpu.sync_copy(pos_hbm, pos_v)
        p = [pos_v[pl.ds(q * LANES, LANES)] for q in range(EMB // LANES)]
        pltpu.sync_copy(tok_hbm.at[pl.ds(base128, idx_rows)], idx_all)

        def idx_slice(c):
            return idx_all.at[c // 2, pl.ds((c % 2) * CHUNK, CHUNK)]

        def gather(c, b):
            pltpu.async_copy(table_hbm.at[idx_slice(c)], bufs.at[b], gsem.at[b])

        def gather_wait(c, b):
            pltpu.make_async_copy(
                table_hbm.at[idx_slice(c)], bufs.at[b], gsem.at[b]).wait()

        def store(c, b):
            pltpu.async_copy(
                bufs.at[b], out_hbm.at[pl.ds(base_row + c * CHUNK, CHUNK)],
                ssem.at[b])

        def store_wait(b):
            pltpu.make_async_copy(
                bufs.at[b], out_hbm.at[pl.ds(base_row, CHUNK)],
                ssem.at[b]).wait()

        for f in range(AHEAD):
            gather(f, f)

        def outer_body(o, carry):
            for b in range(NBUF):
                t = o * NBUF + b
                f = t + AHEAD
                fb = (b + AHEAD) % NBUF

                @pl.when(t > 1)
                def _wait_store():
                    store_wait(fb)

                @pl.when(f < chunks)
                def _issue_gather():
                    gather(f, fb)

                gather_wait(t, b)

                def row_body(i, c):
                    for q in range(EMB // LANES):
                        sl = (b, i, pl.ds(q * LANES, LANES))
                        bufs[sl] = bufs[sl] + p[q]
                    return c

                lax.fori_loop(0, CHUNK, row_body, 0)
                store(t, b)
            return carry

        lax.fori_loop(0, outer, outer_body, 0)

        for k in range(chunks - (NBUF - AHEAD), chunks):
            store_wait(k % NBUF)

    return emb_kernel


def kernel(tokens, table):
    t_dim, b_dim = tokens.shape
    n_rows = t_dim * b_dim
    n_words, emb = table.shape

    den = jnp.exp(-jnp.arange(0, emb, 2, dtype=jnp.float32) * math.log(10000.0) / emb)
    pos = jnp.zeros((emb,), dtype=jnp.float32)
    pos = pos.at[0::2].set(jnp.sin(t_dim * den))
    pos = pos.at[1::2].set(jnp.cos(t_dim * den))

    tok = tokens.reshape(n_rows // 128, 128).astype(jnp.int32)
    table_p = jnp.pad(table, ((0, 0), (0, ROW - emb)))
    out = _build(n_rows, n_words)(tok, table_p, pos)
    return out[:, :emb].reshape(t_dim, b_dim, emb)
